# baseline (speedup 1.0000x reference)
import functools

import jax
import jax.numpy as jnp
from jax import lax
from jax.experimental import pallas as pl
from jax.experimental.pallas import tpu as pltpu
from jax.experimental.pallas import tpu_sc as plsc

N = 10000
E = 160000
D_IN = 256
D_H = 512
D_OUT = 128

NC = 2
NS = 16
HALF = D_IN // NC
BLK = 80
E_PAD = 163840
EPS = E_PAD // NS
NBLK = EPS // BLK
N_PAD = 10240
ROWS_PS = N_PAD // NS


def _sc_body(x0_hbm, x1_hbm, src_hbm, dst_hbm, w_hbm, out_hbm,
             acc_sh, srcv, dstv, wv, rows,
             gsem0, gsem1, ssem0, ssem1, isem0, isem1, isem2, isem3):
    c = lax.axis_index("c")
    s = lax.axis_index("s")
    gsem = (gsem0, gsem1)
    ssem = (ssem0, ssem1)
    isem = (isem0, isem1, isem2, isem3)

    z0 = rows.at[0]

    def zrow(jj, _):
        z16 = jnp.zeros((16,), jnp.float32)
        for f in range(HALF // 16):
            z0[jj, pl.ds(f * 16, 16)] = z16
        return 0

    lax.fori_loop(0, BLK, zrow, 0)
    for k in range(ROWS_PS // BLK):
        pltpu.sync_copy(z0, acc_sh.at[pl.ds(s * ROWS_PS + k * BLK, BLK)])
    plsc.subcore_barrier()

    def _run_half(x_hbm):
        def fetch_idx(j, slot):
            eoff = s * EPS + j * BLK
            pltpu.async_copy(src_hbm.at[pl.ds(eoff, BLK)],
                             srcv.at[slot], isem[slot])
            pltpu.async_copy(dst_hbm.at[pl.ds(eoff, BLK)],
                             dstv.at[slot], isem[slot])
            pltpu.async_copy(w_hbm.at[pl.ds(eoff, BLK)],
                             wv.at[slot], isem[slot])

        def wait_idx(j, slot):
            eoff = s * EPS + j * BLK
            pltpu.make_async_copy(src_hbm.at[pl.ds(eoff, BLK)],
                                  srcv.at[slot], isem[slot]).wait()
            pltpu.make_async_copy(dst_hbm.at[pl.ds(eoff, BLK)],
                                  dstv.at[slot], isem[slot]).wait()
            pltpu.make_async_copy(w_hbm.at[pl.ds(eoff, BLK)],
                                  wv.at[slot], isem[slot]).wait()

        def start_gather(slot, b):
            pltpu.async_copy(x_hbm.at[srcv.at[slot]], rows.at[b], gsem[b])

        def wait_gather(slot, b):
            pltpu.make_async_copy(x_hbm.at[srcv.at[slot]], rows.at[b],
                                  gsem[b]).wait()

        def start_scatter(slot, b):
            pltpu.async_copy(rows.at[b], acc_sh.at[dstv.at[slot]], ssem[b],
                             add=True)

        def wait_scatter(slot, b):
            pltpu.make_async_copy(rows.at[b], acc_sh.at[dstv.at[slot]],
                                  ssem[b]).wait()

        def multiply(slot, b):
            rows_b = rows.at[b]
            w_b = wv.at[slot]

            def grp_body(g, _):
                wrow = w_b[pl.ds(g * 16, 16)]
                base = g * 16
                for t in range(16):
                    wvec = jnp.full((16,), wrow[t], jnp.float32)
                    jj = base + t
                    for f in range(HALF // 16):
                        sl = pl.ds(f * 16, 16)
                        rows_b[jj, sl] = rows_b[jj, sl] * wvec
                return 0

            lax.fori_loop(0, BLK // 16, grp_body, 0)

        fetch_idx(0, 0)
        fetch_idx(1, 1)
        wait_idx(0, 0)
        start_gather(0, 0)

        def quad_body(j4, _):
            for k in range(4):
                j = j4 * 4 + k
                b = k % 2
                slot = k

                @pl.when(j + 1 < NBLK)
                def _():
                    wait_idx(j + 1, (k + 1) % 4)

                @pl.when(j >= 1)
                def _():
                    wait_scatter((k + 3) % 4, 1 - b)

                @pl.when(j + 1 < NBLK)
                def _():
                    start_gather((k + 1) % 4, 1 - b)

                @pl.when(j + 2 < NBLK)
                def _():
                    fetch_idx(j + 2, (k + 2) % 4)

                wait_gather(slot, b)
                multiply(slot, b)
                start_scatter(slot, b)
            return 0

        lax.fori_loop(0, NBLK // 4, quad_body, 0)
        wait_scatter((NBLK - 1) % 4, (NBLK - 1) % 2)

    @pl.when(c == 0)
    def _():
        _run_half(x0_hbm)

    @pl.when(c == 1)
    def _():
        _run_half(x1_hbm)

    plsc.subcore_barrier()
    d0 = rows.at[0]
    for k in range(ROWS_PS // BLK):
        off = s * ROWS_PS + k * BLK
        pltpu.sync_copy(acc_sh.at[pl.ds(off, BLK)], d0)
        pltpu.sync_copy(d0, out_hbm.at[c, pl.ds(off, BLK)])


_sc_agg = pl.kernel(
    _sc_body,
    out_type=jax.ShapeDtypeStruct((NC, N_PAD, HALF), jnp.float32),
    mesh=plsc.VectorSubcoreMesh(core_axis_name="c", subcore_axis_name="s"),
    scratch_types=[
        pltpu.VMEM_SHARED((N_PAD, HALF), jnp.float32),
        pltpu.VMEM((4, BLK), jnp.int32),
        pltpu.VMEM((4, BLK), jnp.int32),
        pltpu.VMEM((4, BLK), jnp.float32),
        pltpu.VMEM((2, BLK, HALF), jnp.float32),
    ] + [pltpu.SemaphoreType.DMA] * 8,
)


ROWB = 640
NSTEP = N_PAD // ROWB


def _tc_body(a0_ref, a1_ref, w0_ref, w1_ref, b2_ref, wm_ref, bm_ref,
             out_ref, acc_ref):
    i = pl.program_id(0)
    h = jax.lax.dot_general(a0_ref[0], w0_ref[...], (((1,), (0,)), ((), ())),
                            preferred_element_type=jnp.float32)
    h = h + jax.lax.dot_general(a1_ref[0], w1_ref[...],
                                (((1,), (0,)), ((), ())),
                                preferred_element_type=jnp.float32)
    h = jnp.maximum(h + b2_ref[...], 0.0)
    row = i * ROWB + lax.broadcasted_iota(jnp.int32, (ROWB, D_H), 0)
    h = jnp.where(row < N, h, 0.0)
    part = jnp.sum(h, axis=0, keepdims=True)

    @pl.when(i == 0)
    def _():
        acc_ref[...] = jnp.zeros_like(acc_ref)

    acc_ref[...] += part

    @pl.when(i == NSTEP - 1)
    def _():
        hg = acc_ref[...] * (1.0 / N)
        logits = jax.lax.dot_general(hg, wm_ref[...], (((1,), (1,)), ((), ())),
                                     preferred_element_type=jnp.float32)
        logits = logits + bm_ref[...]
        m = jnp.max(logits, axis=1, keepdims=True)
        z = logits - m
        lse = jnp.log(jnp.sum(jnp.exp(z), axis=1, keepdims=True))
        out_ref[...] = z - lse


def _tc_head(agg, w2t0, w2t1, b2r, wm, bmr):
    return pl.pallas_call(
        _tc_body,
        grid=(NSTEP,),
        in_specs=[
            pl.BlockSpec((1, ROWB, HALF), lambda i: (0, i, 0)),
            pl.BlockSpec((1, ROWB, HALF), lambda i: (1, i, 0)),
            pl.BlockSpec((HALF, D_H), lambda i: (0, 0)),
            pl.BlockSpec((HALF, D_H), lambda i: (0, 0)),
            pl.BlockSpec((1, D_H), lambda i: (0, 0)),
            pl.BlockSpec((D_OUT, D_H), lambda i: (0, 0)),
            pl.BlockSpec((1, D_OUT), lambda i: (0, 0)),
        ],
        out_specs=pl.BlockSpec((1, D_OUT), lambda i: (0, 0)),
        out_shape=jax.ShapeDtypeStruct((1, D_OUT), jnp.float32),
        scratch_shapes=[pltpu.VMEM((1, D_H), jnp.float32)],
    )(agg, agg, w2t0, w2t1, b2r, wm, bmr)


def kernel(x, edge_index, edge_attr, W0, b0, W1, b1, W2, b2, Wm, bm):
    src = edge_index[0]
    dst = edge_index[1]
    w = edge_attr[:, 0]

    pad = E_PAD - E
    spread = (jnp.arange(pad, dtype=jnp.int32) * 37) % N
    src_p = jnp.concatenate([src, spread])
    dst_p = jnp.concatenate([dst, spread])
    w_p = jnp.concatenate([w, jnp.zeros((pad,), jnp.float32)])


    x0 = x[:, :HALF]
    x1 = x[:, HALF:]

    agg = _sc_agg(x0, x1, src_p, dst_p, w_p)

    w2t = W2.T
    out = _tc_head(agg, w2t[:HALF], w2t[HALF:],
                   b2.reshape(1, D_H), Wm, bm.reshape(1, D_OUT))
    return out.reshape(D_OUT)


# speedup vs baseline: 1.0908x; 1.0908x over previous
import functools

import jax
import jax.numpy as jnp
from jax import lax
from jax.experimental import pallas as pl
from jax.experimental.pallas import tpu as pltpu
from jax.experimental.pallas import tpu_sc as plsc

N = 10000
E = 160000
D_IN = 256
D_H = 512
D_OUT = 128

NC = 2
NS = 16
HALF = D_IN // NC
BLK = 128
E_PAD = 163840
EPS = E_PAD // NS
NBLK = EPS // BLK
N_PAD = 10240
ROWS_PS = N_PAD // NS


def _sc_body(x0_hbm, x1_hbm, src_hbm, dst_hbm, w_hbm, out_hbm,
             acc_sh, srcv, dstv, wv, rows,
             gsem0, gsem1, ssem0, ssem1, isem0, isem1, isem2, isem3):
    c = lax.axis_index("c")
    s = lax.axis_index("s")
    gsem = (gsem0, gsem1)
    ssem = (ssem0, ssem1)
    isem = (isem0, isem1, isem2, isem3)

    z0 = rows.at[0]

    def zrow(jj, _):
        z16 = jnp.zeros((16,), jnp.float32)
        for f in range(HALF // 16):
            z0[jj, pl.ds(f * 16, 16)] = z16
        return 0

    lax.fori_loop(0, BLK, zrow, 0)
    for k in range(ROWS_PS // BLK):
        pltpu.sync_copy(z0, acc_sh.at[pl.ds(s * ROWS_PS + k * BLK, BLK)])
    plsc.subcore_barrier()

    def _run_half(x_hbm):
        def fetch_idx(j, slot):
            eoff = s * EPS + j * BLK
            pltpu.async_copy(src_hbm.at[pl.ds(eoff, BLK)],
                             srcv.at[slot], isem[slot])
            pltpu.async_copy(dst_hbm.at[pl.ds(eoff, BLK)],
                             dstv.at[slot], isem[slot])
            pltpu.async_copy(w_hbm.at[pl.ds(eoff, BLK)],
                             wv.at[slot], isem[slot])

        def wait_idx(j, slot):
            eoff = s * EPS + j * BLK
            pltpu.make_async_copy(src_hbm.at[pl.ds(eoff, BLK)],
                                  srcv.at[slot], isem[slot]).wait()
            pltpu.make_async_copy(dst_hbm.at[pl.ds(eoff, BLK)],
                                  dstv.at[slot], isem[slot]).wait()
            pltpu.make_async_copy(w_hbm.at[pl.ds(eoff, BLK)],
                                  wv.at[slot], isem[slot]).wait()

        def start_gather(slot, b):
            pltpu.async_copy(x_hbm.at[srcv.at[slot]], rows.at[b], gsem[b])

        def wait_gather(slot, b):
            pltpu.make_async_copy(x_hbm.at[srcv.at[slot]], rows.at[b],
                                  gsem[b]).wait()

        def start_scatter(slot, b):
            pltpu.async_copy(rows.at[b], acc_sh.at[dstv.at[slot]], ssem[b],
                             add=True)

        def wait_scatter(slot, b):
            pltpu.make_async_copy(rows.at[b], acc_sh.at[dstv.at[slot]],
                                  ssem[b]).wait()

        def multiply(slot, b):
            rows_b = rows.at[b]
            w_b = wv.at[slot]

            def grp_body(g, _):
                wrow = w_b[pl.ds(g * 16, 16)]
                base = g * 16
                for t in range(16):
                    wvec = jnp.full((16,), wrow[t], jnp.float32)
                    jj = base + t
                    for f in range(HALF // 16):
                        sl = pl.ds(f * 16, 16)
                        rows_b[jj, sl] = rows_b[jj, sl] * wvec
                return 0

            lax.fori_loop(0, BLK // 16, grp_body, 0)

        fetch_idx(0, 0)
        fetch_idx(1, 1)
        wait_idx(0, 0)
        start_gather(0, 0)

        def quad_body(j4, _):
            for k in range(4):
                j = j4 * 4 + k
                b = k % 2
                slot = k

                @pl.when(j + 1 < NBLK)
                def _():
                    wait_idx(j + 1, (k + 1) % 4)

                @pl.when(j >= 1)
                def _():
                    wait_scatter((k + 3) % 4, 1 - b)

                @pl.when(j + 1 < NBLK)
                def _():
                    start_gather((k + 1) % 4, 1 - b)

                @pl.when(j + 2 < NBLK)
                def _():
                    fetch_idx(j + 2, (k + 2) % 4)

                wait_gather(slot, b)
                multiply(slot, b)
                start_scatter(slot, b)
            return 0

        lax.fori_loop(0, NBLK // 4, quad_body, 0)
        wait_scatter((NBLK - 1) % 4, (NBLK - 1) % 2)

    @pl.when(c == 0)
    def _():
        _run_half(x0_hbm)

    @pl.when(c == 1)
    def _():
        _run_half(x1_hbm)

    plsc.subcore_barrier()
    nchunk = ROWS_PS // BLK
    lsem = (gsem0, gsem1)
    osem = (ssem0, ssem1)

    def _off(k):
        return s * ROWS_PS + k * BLK

    for k in (0, 1):
        pltpu.async_copy(acc_sh.at[pl.ds(_off(k), BLK)], rows.at[k % 2],
                         lsem[k % 2])
    for k in range(nchunk):
        b = k % 2
        pltpu.make_async_copy(acc_sh.at[pl.ds(_off(k), BLK)], rows.at[b],
                              lsem[b]).wait()
        pltpu.async_copy(rows.at[b], out_hbm.at[c, pl.ds(_off(k), BLK)],
                         osem[b])
        if k + 2 < nchunk:
            pltpu.make_async_copy(rows.at[b],
                                  out_hbm.at[c, pl.ds(_off(k), BLK)],
                                  osem[b]).wait()
            pltpu.async_copy(acc_sh.at[pl.ds(_off(k + 2), BLK)], rows.at[b],
                             lsem[b])
    for k in (nchunk - 2, nchunk - 1):
        b = k % 2
        pltpu.make_async_copy(rows.at[b], out_hbm.at[c, pl.ds(_off(k), BLK)],
                              osem[b]).wait()


_sc_agg = pl.kernel(
    _sc_body,
    out_type=jax.ShapeDtypeStruct((NC, N_PAD, HALF), jnp.float32),
    mesh=plsc.VectorSubcoreMesh(core_axis_name="c", subcore_axis_name="s"),
    scratch_types=[
        pltpu.VMEM_SHARED((N_PAD, HALF), jnp.float32),
        pltpu.VMEM((4, BLK), jnp.int32),
        pltpu.VMEM((4, BLK), jnp.int32),
        pltpu.VMEM((4, BLK), jnp.float32),
        pltpu.VMEM((2, BLK, HALF), jnp.float32),
    ] + [pltpu.SemaphoreType.DMA] * 8,
)


ROWB = 1280
NSTEP = N_PAD // ROWB


def _tc_body(a0_ref, a1_ref, w0_ref, w1_ref, b2_ref, wm_ref, bm_ref,
             out_ref, acc_ref):
    i = pl.program_id(0)
    h = jax.lax.dot_general(a0_ref[0], w0_ref[...], (((1,), (0,)), ((), ())),
                            preferred_element_type=jnp.float32)
    h = h + jax.lax.dot_general(a1_ref[0], w1_ref[...],
                                (((1,), (0,)), ((), ())),
                                preferred_element_type=jnp.float32)
    h = jnp.maximum(h + b2_ref[...], 0.0)
    row = i * ROWB + lax.broadcasted_iota(jnp.int32, (ROWB, D_H), 0)
    h = jnp.where(row < N, h, 0.0)
    part = jnp.sum(h, axis=0, keepdims=True)

    @pl.when(i == 0)
    def _():
        acc_ref[...] = jnp.zeros_like(acc_ref)

    acc_ref[...] += part

    @pl.when(i == NSTEP - 1)
    def _():
        hg = acc_ref[...] * (1.0 / N)
        logits = jax.lax.dot_general(hg, wm_ref[...], (((1,), (1,)), ((), ())),
                                     preferred_element_type=jnp.float32)
        logits = logits + bm_ref[...]
        m = jnp.max(logits, axis=1, keepdims=True)
        z = logits - m
        lse = jnp.log(jnp.sum(jnp.exp(z), axis=1, keepdims=True))
        out_ref[...] = z - lse


def _tc_head(agg, w2t0, w2t1, b2r, wm, bmr):
    return pl.pallas_call(
        _tc_body,
        grid=(NSTEP,),
        in_specs=[
            pl.BlockSpec((1, ROWB, HALF), lambda i: (0, i, 0)),
            pl.BlockSpec((1, ROWB, HALF), lambda i: (1, i, 0)),
            pl.BlockSpec((HALF, D_H), lambda i: (0, 0)),
            pl.BlockSpec((HALF, D_H), lambda i: (0, 0)),
            pl.BlockSpec((1, D_H), lambda i: (0, 0)),
            pl.BlockSpec((D_OUT, D_H), lambda i: (0, 0)),
            pl.BlockSpec((1, D_OUT), lambda i: (0, 0)),
        ],
        out_specs=pl.BlockSpec((1, D_OUT), lambda i: (0, 0)),
        out_shape=jax.ShapeDtypeStruct((1, D_OUT), jnp.float32),
        scratch_shapes=[pltpu.VMEM((1, D_H), jnp.float32)],
    )(agg, agg, w2t0, w2t1, b2r, wm, bmr)


def kernel(x, edge_index, edge_attr, W0, b0, W1, b1, W2, b2, Wm, bm):
    src = edge_index[0]
    dst = edge_index[1]
    w = edge_attr[:, 0]

    pad = E_PAD - E
    spread = (jnp.arange(pad, dtype=jnp.int32) * 37) % N
    src_p = jnp.concatenate([src, spread])
    dst_p = jnp.concatenate([dst, spread])
    w_p = jnp.concatenate([w, jnp.zeros((pad,), jnp.float32)])


    x0 = x[:, :HALF]
    x1 = x[:, HALF:]

    agg = _sc_agg(x0, x1, src_p, dst_p, w_p)

    w2t = W2.T
    out = _tc_head(agg, w2t[:HALF], w2t[HALF:],
                   b2.reshape(1, D_H), Wm, bm.reshape(1, D_OUT))
    return out.reshape(D_OUT)


# speedup vs baseline: 1.1047x; 1.0127x over previous
import functools

import jax
import jax.numpy as jnp
from jax import lax
from jax.experimental import pallas as pl
from jax.experimental.pallas import tpu as pltpu
from jax.experimental.pallas import tpu_sc as plsc

N = 10000
E = 160000
D_IN = 256
D_H = 512
D_OUT = 128

NC = 2
NS = 16
HALF = D_IN // NC
BLK = 128
E_PAD = 163840
EPS = E_PAD // NS
NBLK = EPS // BLK
N_PAD = 10240
ROWS_PS = N_PAD // NS


def _sc_body(x0_hbm, x1_hbm, src_hbm, dst_hbm, w_hbm, out_hbm,
             acc_sh, srcv, dstv, wv, rows,
             gsem0, gsem1, ssem0, ssem1, isem0, isem1, isem2, isem3):
    c = lax.axis_index("c")
    s = lax.axis_index("s")
    gsem = (gsem0, gsem1)
    ssem = (ssem0, ssem1)
    isem = (isem0, isem1, isem2, isem3)

    def _run_half(x_hbm):
        def fetch_idx(j, slot):
            eoff = s * EPS + j * BLK
            pltpu.async_copy(src_hbm.at[pl.ds(eoff, BLK)],
                             srcv.at[slot], isem[slot])
            pltpu.async_copy(dst_hbm.at[pl.ds(eoff, BLK)],
                             dstv.at[slot], isem[slot])
            pltpu.async_copy(w_hbm.at[pl.ds(eoff, BLK)],
                             wv.at[slot], isem[slot])

        def wait_idx(j, slot):
            eoff = s * EPS + j * BLK
            pltpu.make_async_copy(src_hbm.at[pl.ds(eoff, BLK)],
                                  srcv.at[slot], isem[slot]).wait()
            pltpu.make_async_copy(dst_hbm.at[pl.ds(eoff, BLK)],
                                  dstv.at[slot], isem[slot]).wait()
            pltpu.make_async_copy(w_hbm.at[pl.ds(eoff, BLK)],
                                  wv.at[slot], isem[slot]).wait()

        def start_gather(slot, b):
            pltpu.async_copy(x_hbm.at[srcv.at[slot]], rows.at[b], gsem[b])

        def wait_gather(slot, b):
            pltpu.make_async_copy(x_hbm.at[srcv.at[slot]], rows.at[b],
                                  gsem[b]).wait()

        def start_scatter(slot, b):
            pltpu.async_copy(rows.at[b], acc_sh.at[dstv.at[slot]], ssem[b],
                             add=True)

        def wait_scatter(slot, b):
            pltpu.make_async_copy(rows.at[b], acc_sh.at[dstv.at[slot]],
                                  ssem[b]).wait()

        def multiply(slot, b):
            rows_b = rows.at[b]
            w_b = wv.at[slot]

            def grp_body(g, _):
                wrow = w_b[pl.ds(g * 16, 16)]
                base = g * 16
                for t in range(16):
                    wvec = jnp.full((16,), wrow[t], jnp.float32)
                    jj = base + t
                    for f in range(HALF // 16):
                        sl = pl.ds(f * 16, 16)
                        rows_b[jj, sl] = rows_b[jj, sl] * wvec
                return 0

            lax.fori_loop(0, BLK // 16, grp_body, 0)

        fetch_idx(0, 0)
        fetch_idx(1, 1)
        z1 = rows.at[1]

        def zrow(jj, _):
            z16 = jnp.zeros((16,), jnp.float32)
            for f in range(HALF // 16):
                z1[jj, pl.ds(f * 16, 16)] = z16
            return 0

        lax.fori_loop(0, BLK, zrow, 0)
        wait_idx(0, 0)
        start_gather(0, 0)
        for k in range(ROWS_PS // BLK):
            pltpu.async_copy(z1, acc_sh.at[pl.ds(s * ROWS_PS + k * BLK, BLK)],
                             ssem1)
        for k in range(ROWS_PS // BLK):
            pltpu.make_async_copy(
                z1, acc_sh.at[pl.ds(s * ROWS_PS + k * BLK, BLK)],
                ssem1).wait()
        plsc.subcore_barrier()

        def quad_body(j4, _):
            for k in range(4):
                j = j4 * 4 + k
                b = k % 2
                slot = k

                @pl.when(j + 1 < NBLK)
                def _():
                    wait_idx(j + 1, (k + 1) % 4)

                @pl.when(j >= 1)
                def _():
                    wait_scatter((k + 3) % 4, 1 - b)

                @pl.when(j + 1 < NBLK)
                def _():
                    start_gather((k + 1) % 4, 1 - b)

                @pl.when(j + 2 < NBLK)
                def _():
                    fetch_idx(j + 2, (k + 2) % 4)

                wait_gather(slot, b)
                multiply(slot, b)
                start_scatter(slot, b)
            return 0

        lax.fori_loop(0, NBLK // 4, quad_body, 0)
        wait_scatter((NBLK - 1) % 4, (NBLK - 1) % 2)

    @pl.when(c == 0)
    def _():
        _run_half(x0_hbm)

    @pl.when(c == 1)
    def _():
        _run_half(x1_hbm)

    plsc.subcore_barrier()
    nchunk = ROWS_PS // BLK
    lsem = (gsem0, gsem1)
    osem = (ssem0, ssem1)

    def _off(k):
        return s * ROWS_PS + k * BLK

    for k in (0, 1):
        pltpu.async_copy(acc_sh.at[pl.ds(_off(k), BLK)], rows.at[k % 2],
                         lsem[k % 2])
    for k in range(nchunk):
        b = k % 2
        pltpu.make_async_copy(acc_sh.at[pl.ds(_off(k), BLK)], rows.at[b],
                              lsem[b]).wait()
        pltpu.async_copy(rows.at[b], out_hbm.at[c, pl.ds(_off(k), BLK)],
                         osem[b])
        if k + 2 < nchunk:
            pltpu.make_async_copy(rows.at[b],
                                  out_hbm.at[c, pl.ds(_off(k), BLK)],
                                  osem[b]).wait()
            pltpu.async_copy(acc_sh.at[pl.ds(_off(k + 2), BLK)], rows.at[b],
                             lsem[b])
    for k in (nchunk - 2, nchunk - 1):
        b = k % 2
        pltpu.make_async_copy(rows.at[b], out_hbm.at[c, pl.ds(_off(k), BLK)],
                              osem[b]).wait()


_sc_agg = pl.kernel(
    _sc_body,
    out_type=jax.ShapeDtypeStruct((NC, N_PAD, HALF), jnp.float32),
    mesh=plsc.VectorSubcoreMesh(core_axis_name="c", subcore_axis_name="s"),
    scratch_types=[
        pltpu.VMEM_SHARED((N_PAD, HALF), jnp.float32),
        pltpu.VMEM((4, BLK), jnp.int32),
        pltpu.VMEM((4, BLK), jnp.int32),
        pltpu.VMEM((4, BLK), jnp.float32),
        pltpu.VMEM((2, BLK, HALF), jnp.float32),
    ] + [pltpu.SemaphoreType.DMA] * 8,
)


ROWB = 1280
NSTEP = N_PAD // ROWB


def _tc_body(a0_ref, a1_ref, w0_ref, w1_ref, b2_ref, wm_ref, bm_ref,
             out_ref, acc_ref):
    i = pl.program_id(0)
    h = jax.lax.dot_general(a0_ref[0], w0_ref[...], (((1,), (0,)), ((), ())),
                            preferred_element_type=jnp.float32)
    h = h + jax.lax.dot_general(a1_ref[0], w1_ref[...],
                                (((1,), (0,)), ((), ())),
                                preferred_element_type=jnp.float32)
    h = jnp.maximum(h + b2_ref[...], 0.0)
    row = i * ROWB + lax.broadcasted_iota(jnp.int32, (ROWB, D_H), 0)
    h = jnp.where(row < N, h, 0.0)
    part = jnp.sum(h, axis=0, keepdims=True)

    @pl.when(i == 0)
    def _():
        acc_ref[...] = jnp.zeros_like(acc_ref)

    acc_ref[...] += part

    @pl.when(i == NSTEP - 1)
    def _():
        hg = acc_ref[...] * (1.0 / N)
        logits = jax.lax.dot_general(hg, wm_ref[...], (((1,), (1,)), ((), ())),
                                     preferred_element_type=jnp.float32)
        logits = logits + bm_ref[...]
        m = jnp.max(logits, axis=1, keepdims=True)
        z = logits - m
        lse = jnp.log(jnp.sum(jnp.exp(z), axis=1, keepdims=True))
        out_ref[...] = z - lse


def _tc_head(agg, w2t0, w2t1, b2r, wm, bmr):
    return pl.pallas_call(
        _tc_body,
        grid=(NSTEP,),
        in_specs=[
            pl.BlockSpec((1, ROWB, HALF), lambda i: (0, i, 0)),
            pl.BlockSpec((1, ROWB, HALF), lambda i: (1, i, 0)),
            pl.BlockSpec((HALF, D_H), lambda i: (0, 0)),
            pl.BlockSpec((HALF, D_H), lambda i: (0, 0)),
            pl.BlockSpec((1, D_H), lambda i: (0, 0)),
            pl.BlockSpec((D_OUT, D_H), lambda i: (0, 0)),
            pl.BlockSpec((1, D_OUT), lambda i: (0, 0)),
        ],
        out_specs=pl.BlockSpec((1, D_OUT), lambda i: (0, 0)),
        out_shape=jax.ShapeDtypeStruct((1, D_OUT), jnp.float32),
        scratch_shapes=[pltpu.VMEM((1, D_H), jnp.float32)],
    )(agg, agg, w2t0, w2t1, b2r, wm, bmr)


def kernel(x, edge_index, edge_attr, W0, b0, W1, b1, W2, b2, Wm, bm):
    src = edge_index[0]
    dst = edge_index[1]
    w = edge_attr[:, 0]

    pad = E_PAD - E
    spread = (jnp.arange(pad, dtype=jnp.int32) * 37) % N
    src_p = jnp.concatenate([src, spread])
    dst_p = jnp.concatenate([dst, spread])
    w_p = jnp.concatenate([w, jnp.zeros((pad,), jnp.float32)])


    x0 = x[:, :HALF]
    x1 = x[:, HALF:]

    agg = _sc_agg(x0, x1, src_p, dst_p, w_p)

    w2t = W2.T
    out = _tc_head(agg, w2t[:HALF], w2t[HALF:],
                   b2.reshape(1, D_H), Wm, bm.reshape(1, D_OUT))
    return out.reshape(D_OUT)
